# baseline (speedup 1.0000x reference)
import functools

import jax
import jax.numpy as jnp
from jax import lax
from jax.experimental import pallas as pl
from jax.experimental.pallas import tpu as pltpu
from jax.experimental.pallas import tpu_sc as plsc

CLS_N = 100000
FEAT_N = 64
BATCH_N = 16384

NC = 2
NS = 16
LANES = 16
NW = NC * NS
BPW = BATCH_N // NW
YS_ROWS = BATCH_N // 128
ROWS_PER_TILE = YS_ROWS // NW
ROWS_PER_SUB = YS_ROWS // NS
HIST_PER_SUB = 6272
V_PAD = NS * HIST_PER_SUB

_MESH = plsc.VectorSubcoreMesh(core_axis_name="c", subcore_axis_name="s")
_CP = pltpu.CompilerParams(use_tc_tiling_on_sc=True)


def _sc_gather_and_count(ys2, center):
  @functools.partial(
      pl.kernel,
      out_type=[
          jax.ShapeDtypeStruct((BATCH_N, FEAT_N), jnp.float32),
          jax.ShapeDtypeStruct((BATCH_N,), jnp.float32),
      ],
      mesh=_MESH,
      compiler_params=_CP,
      scratch_types=[
          pltpu.VMEM((ROWS_PER_TILE, 128), jnp.int32),
          pltpu.VMEM((ROWS_PER_SUB, 128), jnp.int32),
          pltpu.VMEM((BPW,), jnp.float32),
          pltpu.VMEM((128,), jnp.float32),
          pltpu.VMEM((HIST_PER_SUB,), jnp.float32),
          pltpu.VMEM_SHARED((V_PAD,), jnp.float32),
          pltpu.SemaphoreType.DMA,
          pltpu.SemaphoreType.DMA,
      ],
  )
  def sc_kernel(ys_hbm, center_hbm, out_g, out_c,
                idx_v, hidx_v, cnt_v, ones_v, zeros_v, hist, gsem, hsem):
    cid = lax.axis_index("c")
    sid = lax.axis_index("s")
    wid = sid * NC + cid

    pltpu.sync_copy(ys_hbm.at[pl.ds(wid * ROWS_PER_TILE, ROWS_PER_TILE)],
                    idx_v)

    for r in range(ROWS_PER_TILE):
      @pl.loop(0, 128, step=LANES)
      def _(j, r=r):
        v = idx_v[r, pl.ds(j, LANES)]
        for k in range(LANES):
          y = v[k]
          pltpu.async_copy(
              center_hbm.at[pl.ds(y, 1)],
              out_g.at[pl.ds(wid * BPW + r * 128 + j + k, 1)], gsem)

    pltpu.sync_copy(ys_hbm.at[pl.ds(sid * ROWS_PER_SUB, ROWS_PER_SUB)],
                    hidx_v)

    @pl.loop(0, 128, step=LANES)
    def _(i):
      ones_v[pl.ds(i, LANES)] = jnp.ones((LANES,), jnp.float32)

    @pl.loop(0, HIST_PER_SUB, step=LANES)
    def _(i):
      zeros_v[pl.ds(i, LANES)] = jnp.zeros((LANES,), jnp.float32)

    pltpu.sync_copy(zeros_v, hist.at[pl.ds(sid * HIST_PER_SUB, HIST_PER_SUB)])
    plsc.subcore_barrier()

    hadds = [pltpu.async_copy(ones_v, hist.at[hidx_v.at[c]], hsem, add=True)
             for c in range(ROWS_PER_SUB)]
    for cp in hadds:
      cp.wait()
    plsc.subcore_barrier()

    cgets = [pltpu.async_copy(hist.at[idx_v.at[c]],
                              cnt_v.at[pl.ds(c * 128, 128)], hsem)
             for c in range(ROWS_PER_TILE)]
    for cp in cgets:
      cp.wait()

    pltpu.make_async_copy(center_hbm.at[pl.ds(0, BPW)],
                          out_g.at[pl.ds(wid * BPW, BPW)], gsem).wait()

    pltpu.sync_copy(cnt_v, out_c.at[pl.ds(wid * BPW, BPW)])

  return sc_kernel(ys2, center)


_TC_BLK = 4096


def _tc_loss(xs, g, cnt):
  def body(xs_ref, g_ref, cnt_ref, out_ref):
    x = xs_ref[...]
    n2 = jnp.sum(x * x, axis=1, keepdims=True)
    xn = x / jnp.maximum(jnp.sqrt(n2), 1e-12)
    d = xn - g_ref[...]
    ssq = jnp.sum(d * d, axis=1, keepdims=True)
    c = cnt_ref[...].reshape(_TC_BLK, 1)
    s = jnp.sum(ssq / c)

    @pl.when(pl.program_id(0) == 0)
    def _():
      out_ref[...] = jnp.zeros_like(out_ref)

    out_ref[...] += s

  out = pl.pallas_call(
      body,
      grid=(BATCH_N // _TC_BLK,),
      in_specs=[
          pl.BlockSpec((_TC_BLK, FEAT_N), lambda i: (i, 0)),
          pl.BlockSpec((_TC_BLK, FEAT_N), lambda i: (i, 0)),
          pl.BlockSpec((_TC_BLK,), lambda i: (i,)),
      ],
      out_specs=pl.BlockSpec((1, 1), lambda i: (0, 0)),
      out_shape=jax.ShapeDtypeStruct((1, 1), jnp.float32),
  )(xs, g, cnt)
  return out[0, 0]


@jax.jit
def kernel(xs, ys, center):
  ys2 = ys.astype(jnp.int32).reshape(YS_ROWS, 128)
  g, cnt = _sc_gather_and_count(ys2, center)
  return _tc_loss(xs, g, cnt)


# speedup vs baseline: 4.1433x; 4.1433x over previous
import functools

import jax
import jax.numpy as jnp
from jax import lax
from jax.experimental import pallas as pl
from jax.experimental.pallas import tpu as pltpu
from jax.experimental.pallas import tpu_sc as plsc

CLS_N = 100000
FEAT_N = 64
BATCH_N = 16384

NC = 2
NS = 16
LANES = 16
NW = NC * NS
BPW = BATCH_N // NW
YS_ROWS = BATCH_N // 128
ROWS_PER_TILE = YS_ROWS // NW
ROWS_PER_SUB = YS_ROWS // NS
HIST_PER_SUB = 6272
V_PAD = NS * HIST_PER_SUB

_MESH = plsc.VectorSubcoreMesh(core_axis_name="c", subcore_axis_name="s")
_CP = pltpu.CompilerParams(use_tc_tiling_on_sc=True)


def _sc_gather_and_count(ys2, center):
  @functools.partial(
      pl.kernel,
      out_type=[
          jax.ShapeDtypeStruct((BATCH_N, FEAT_N), jnp.float32),
          jax.ShapeDtypeStruct((BATCH_N,), jnp.float32),
      ],
      mesh=_MESH,
      compiler_params=_CP,
      scratch_types=[
          pltpu.VMEM((ROWS_PER_TILE, 128), jnp.int32),
          pltpu.VMEM((ROWS_PER_SUB, 128), jnp.int32),
          pltpu.VMEM((BPW, FEAT_N), jnp.float32),
          pltpu.VMEM((BPW,), jnp.float32),
          pltpu.VMEM((128,), jnp.float32),
          pltpu.VMEM((HIST_PER_SUB,), jnp.float32),
          pltpu.VMEM_SHARED((V_PAD,), jnp.float32),
          pltpu.SemaphoreType.DMA,
          pltpu.SemaphoreType.DMA,
      ],
  )
  def sc_kernel(ys_hbm, center_hbm, out_g, out_c,
                idx_v, hidx_v, rows_v, cnt_v, ones_v, zeros_v, hist, gsem,
                hsem):
    cid = lax.axis_index("c")
    sid = lax.axis_index("s")
    wid = sid * NC + cid

    pltpu.sync_copy(ys_hbm.at[pl.ds(wid * ROWS_PER_TILE, ROWS_PER_TILE)],
                    idx_v)

    for r in range(ROWS_PER_TILE):
      @pl.loop(0, 128, step=LANES)
      def _(j, r=r):
        v = idx_v[r, pl.ds(j, LANES)]
        for k in range(LANES):
          y = v[k]
          pltpu.async_copy(center_hbm.at[pl.ds(y, 1)],
                           rows_v.at[pl.ds(r * 128 + j + k, 1)], gsem)

    pltpu.sync_copy(ys_hbm.at[pl.ds(sid * ROWS_PER_SUB, ROWS_PER_SUB)],
                    hidx_v)

    @pl.loop(0, 128, step=LANES)
    def _(i):
      ones_v[pl.ds(i, LANES)] = jnp.ones((LANES,), jnp.float32)

    @pl.loop(0, HIST_PER_SUB, step=LANES)
    def _(i):
      zeros_v[pl.ds(i, LANES)] = jnp.zeros((LANES,), jnp.float32)

    pltpu.sync_copy(zeros_v, hist.at[pl.ds(sid * HIST_PER_SUB, HIST_PER_SUB)])
    plsc.subcore_barrier()

    hadds = [pltpu.async_copy(ones_v, hist.at[hidx_v.at[c]], hsem, add=True)
             for c in range(ROWS_PER_SUB)]
    for cp in hadds:
      cp.wait()
    plsc.subcore_barrier()

    cgets = [pltpu.async_copy(hist.at[idx_v.at[c]],
                              cnt_v.at[pl.ds(c * 128, 128)], hsem)
             for c in range(ROWS_PER_TILE)]
    for cp in cgets:
      cp.wait()

    pltpu.make_async_copy(center_hbm.at[pl.ds(0, BPW)], rows_v, gsem).wait()

    pltpu.sync_copy(rows_v, out_g.at[pl.ds(wid * BPW, BPW)])
    pltpu.sync_copy(cnt_v, out_c.at[pl.ds(wid * BPW, BPW)])

  return sc_kernel(ys2, center)


_TC_BLK = 4096


def _tc_loss(xs, g, cnt):
  def body(xs_ref, g_ref, cnt_ref, out_ref):
    x = xs_ref[...]
    n2 = jnp.sum(x * x, axis=1, keepdims=True)
    xn = x / jnp.maximum(jnp.sqrt(n2), 1e-12)
    d = xn - g_ref[...]
    ssq = jnp.sum(d * d, axis=1, keepdims=True)
    c = cnt_ref[...].reshape(_TC_BLK, 1)
    s = jnp.sum(ssq / c)

    @pl.when(pl.program_id(0) == 0)
    def _():
      out_ref[...] = jnp.zeros_like(out_ref)

    out_ref[...] += s

  out = pl.pallas_call(
      body,
      grid=(BATCH_N // _TC_BLK,),
      in_specs=[
          pl.BlockSpec((_TC_BLK, FEAT_N), lambda i: (i, 0)),
          pl.BlockSpec((_TC_BLK, FEAT_N), lambda i: (i, 0)),
          pl.BlockSpec((_TC_BLK,), lambda i: (i,)),
      ],
      out_specs=pl.BlockSpec((1, 1), lambda i: (0, 0)),
      out_shape=jax.ShapeDtypeStruct((1, 1), jnp.float32),
  )(xs, g, cnt)
  return out[0, 0]


@jax.jit
def kernel(xs, ys, center):
  ys2 = ys.astype(jnp.int32).reshape(YS_ROWS, 128)
  g, cnt = _sc_gather_and_count(ys2, center)
  return _tc_loss(xs, g, cnt)
